# Initial kernel scaffold; baseline (speedup 1.0000x reference)
import jax
import jax.numpy as jnp
from jax.experimental import pallas as pl


def kernel(x, edge_attr, group_emb, hemi_emb):
    raise NotImplementedError("write your pallas kernel here")


# speedup vs baseline: 3.1590x; 3.1590x over previous
import jax
import jax.numpy as jnp
from jax.experimental import pallas as pl

_BLOCK = 2000


def _encode_kernel(x_ref, group_ref, hemi_ref, o_ref):
    block = x_ref.shape[0]
    r0 = pl.program_id(0) * block
    rows = r0 + jax.lax.broadcasted_iota(jnp.int32, (block, 2), 0)

    gid = jnp.where((rows < 8000) & (rows % 1000 < 8), rows // 1000, 0)
    func = jnp.broadcast_to(group_ref[0:1, :], (block, 2))
    for g in range(1, 8):
        func = jnp.where(gid == g, group_ref[g : g + 1, :], func)

    hemi = jnp.where(
        (rows & 1) == 1,
        hemi_ref[1:2, :],
        jnp.broadcast_to(hemi_ref[0:1, :], (block, 2)),
    )

    o_ref[:, 0:128] = x_ref[...]
    o_ref[:, 128:132] = jnp.concatenate([func, hemi], axis=1)


def kernel(x, edge_attr, group_emb, hemi_emb):
    n, d = x.shape
    grid = n // _BLOCK
    x_out = pl.pallas_call(
        _encode_kernel,
        grid=(grid,),
        in_specs=[
            pl.BlockSpec((_BLOCK, d), lambda i: (i, 0)),
            pl.BlockSpec(group_emb.shape, lambda i: (0, 0)),
            pl.BlockSpec(hemi_emb.shape, lambda i: (0, 0)),
        ],
        out_specs=pl.BlockSpec((_BLOCK, d + 4), lambda i: (i, 0)),
        out_shape=jax.ShapeDtypeStruct((n, d + 4), x.dtype),
    )(x, group_emb, hemi_emb)
    return (x_out, edge_attr.astype(jnp.float32))


# speedup vs baseline: 3.3160x; 1.0497x over previous
import jax
import jax.numpy as jnp
from jax.experimental import pallas as pl
from jax.experimental.pallas import tpu as pltpu

_BLOCK = 4000


def _encode_kernel(x_ref, group_ref, hemi_ref, o_ref):
    block = x_ref.shape[0]
    r0 = pl.program_id(0) * block
    rows = r0 + jax.lax.broadcasted_iota(jnp.int32, (block, 2), 0)

    gid = jnp.where((rows < 8000) & (rows % 1000 < 8), rows // 1000, 0)
    func = jnp.broadcast_to(group_ref[0:1, :], (block, 2))
    for g in range(1, 8):
        func = jnp.where(gid == g, group_ref[g : g + 1, :], func)

    hemi = jnp.where(
        (rows & 1) == 1,
        hemi_ref[1:2, :],
        jnp.broadcast_to(hemi_ref[0:1, :], (block, 2)),
    )

    o_ref[:, 0:128] = x_ref[...]
    o_ref[:, 128:132] = jnp.concatenate([func, hemi], axis=1)


def kernel(x, edge_attr, group_emb, hemi_emb):
    n, d = x.shape
    grid = n // _BLOCK
    x_out = pl.pallas_call(
        _encode_kernel,
        grid=(grid,),
        in_specs=[
            pl.BlockSpec((_BLOCK, d), lambda i: (i, 0)),
            pl.BlockSpec(group_emb.shape, lambda i: (0, 0)),
            pl.BlockSpec(hemi_emb.shape, lambda i: (0, 0)),
        ],
        out_specs=pl.BlockSpec((_BLOCK, d + 4), lambda i: (i, 0)),
        out_shape=jax.ShapeDtypeStruct((n, d + 4), x.dtype),
        compiler_params=pltpu.CompilerParams(
            dimension_semantics=("parallel",),
        ),
    )(x, group_emb, hemi_emb)
    return (x_out, edge_attr.astype(jnp.float32))


# speedup vs baseline: 3.6952x; 1.1144x over previous
import jax
import jax.numpy as jnp
from jax.experimental import pallas as pl
from jax.experimental.pallas import tpu as pltpu

_BLOCK = 4000


def _encode_kernel(x_ref, group_ref, hemi_ref, o_ref):
    block = x_ref.shape[0]
    r0 = pl.program_id(0) * block
    o_ref[:, 0:128] = x_ref[...]

    even = jnp.concatenate([group_ref[0:1, :], hemi_ref[0:1, :]], axis=1)
    odd = jnp.concatenate([group_ref[0:1, :], hemi_ref[1:2, :]], axis=1)
    rows = r0 + jax.lax.broadcasted_iota(jnp.int32, (block, 4), 0)
    extra = jnp.where((rows & 1) == 1, odd, jnp.broadcast_to(even, (block, 4)))

    @pl.when(r0 >= 8000)
    def _():
        o_ref[:, 128:132] = extra

    @pl.when(r0 < 8000)
    def _():
        col = jax.lax.broadcasted_iota(jnp.int32, (block, 4), 1)
        gid = jnp.where(rows % 1000 < 8, rows // 1000, 0)
        e = extra
        for g in range(1, 8):
            gval = jnp.concatenate([group_ref[g : g + 1, :]] * 2, axis=1)
            e = jnp.where((gid == g) & (col < 2), gval, e)
        o_ref[:, 128:132] = e


def kernel(x, edge_attr, group_emb, hemi_emb):
    n, d = x.shape
    grid = n // _BLOCK
    x_out = pl.pallas_call(
        _encode_kernel,
        grid=(grid,),
        in_specs=[
            pl.BlockSpec((_BLOCK, d), lambda i: (i, 0)),
            pl.BlockSpec(group_emb.shape, lambda i: (0, 0)),
            pl.BlockSpec(hemi_emb.shape, lambda i: (0, 0)),
        ],
        out_specs=pl.BlockSpec((_BLOCK, d + 4), lambda i: (i, 0)),
        out_shape=jax.ShapeDtypeStruct((n, d + 4), x.dtype),
        compiler_params=pltpu.CompilerParams(
            dimension_semantics=("parallel",),
        ),
    )(x, group_emb, hemi_emb)
    return (x_out, edge_attr.astype(jnp.float32))
